# baseline (speedup 1.0000x reference)
import functools

import jax
import jax.numpy as jnp
from jax import lax
from jax.experimental import pallas as pl
from jax.experimental.pallas import tpu as pltpu
from jax.experimental.pallas import tpu_sc as plsc

B = 16384
F = 128
N_TAB = 1000

NC = 2
NS = 16
NW = NC * NS
ROWS_PER_W = B // NW
CHUNK = 128
NCHUNK = ROWS_PER_W // CHUNK
NBUF = 2
LANES = 16
GROUPS = CHUNK // LANES


STAGE_TILES = 8
STAGE_ROWS = N_TAB // STAGE_TILES


def _sc_body(x_hbm, y_hbm, c_hbm, a_hbm, out_hbm,
             idx0, idx1, idx2, idx3, a_tab, avb,
             xb0, xb1, cb0, cb1, ob0, ob1, c_spm,
             sem_a, sem_i, sem_s, sx0, sx1, sc0, sc1, so0, so1):
    idxs = (idx0, idx1, idx2, idx3)
    xbs = (xb0, xb1)
    cbs = (cb0, cb1)
    obs = (ob0, ob1)
    sxs = (sx0, sx1)
    scs = (sc0, sc1)
    sos = (so0, so1)

    sid = lax.axis_index("s")
    wid = sid * NC + lax.axis_index("c")
    base = wid * ROWS_PER_W

    cp_a = pltpu.async_copy(a_hbm, a_tab, sem_a)
    @pl.when(sid < STAGE_TILES)
    def _stage():
        pltpu.async_copy(
            c_hbm.at[pl.ds(sid * STAGE_ROWS, STAGE_ROWS), :],
            c_spm.at[pl.ds(sid * STAGE_ROWS, STAGE_ROWS), :],
            sem_s).wait()
    cpx = {}
    cpc = {}
    cpo = {}
    for j in range(NBUF):
        cpx[j] = pltpu.async_copy(
            x_hbm.at[pl.ds(base + j * CHUNK, CHUNK), :], xbs[j], sxs[j])
    cpi = [pltpu.async_copy(y_hbm.at[pl.ds(base + s * CHUNK, CHUNK)],
                            idxs[s], sem_i)
           for s in range(NCHUNK)]
    for cp in cpi:
        cp.wait()
    plsc.subcore_barrier()
    for j in range(NBUF):
        cpc[j] = pltpu.async_copy(c_spm.at[idxs[j]], cbs[j], scs[j])
    cp_a.wait()

    for j in range(NCHUNK):
        slot = j % NBUF
        cpx[j].wait()
        cpc[j].wait()
        if j >= NBUF:
            cpo[j - NBUF].wait()
        xbuf, cbuf, obuf, idx = xbs[slot], cbs[slot], obs[slot], idxs[j]

        @plsc.parallel_loop(0, GROUPS, step=1, unroll=1)
        def scale_body(g):
            idxg = idx[pl.ds(g * LANES, LANES)]
            avb[pl.ds(g * LANES, LANES)] = plsc.load_gather(a_tab, [idxg])

        @plsc.parallel_loop(0, CHUNK, step=1, unroll=2)
        def row_body(row):
            sv = plsc.load_gather(
                avb, [jnp.full((LANES,), row, dtype=jnp.int32)])
            for k in range(F // LANES):
                sl = pl.ds(k * LANES, LANES)
                obuf[row, sl] = xbuf[row, sl] * sv + cbuf[row, sl]

        cpo[j] = pltpu.async_copy(
            obuf, out_hbm.at[pl.ds(base + j * CHUNK, CHUNK), :], sos[slot])
        nj = j + NBUF
        if nj < NCHUNK:
            cpx[nj] = pltpu.async_copy(
                x_hbm.at[pl.ds(base + nj * CHUNK, CHUNK), :], xbuf, sxs[slot])
            cpc[nj] = pltpu.async_copy(c_spm.at[idxs[nj]], cbuf, scs[slot])

    for j in range(NCHUNK - NBUF, NCHUNK):
        cpo[j].wait()


@jax.jit
def _run(x, y, c, a_flat):
    mesh = plsc.VectorSubcoreMesh(core_axis_name="c", subcore_axis_name="s")
    fn = functools.partial(
        pl.kernel,
        out_type=jax.ShapeDtypeStruct((B, F), jnp.float32),
        mesh=mesh,
        scratch_types=[
            pltpu.VMEM((CHUNK,), jnp.int32),
            pltpu.VMEM((CHUNK,), jnp.int32),
            pltpu.VMEM((CHUNK,), jnp.int32),
            pltpu.VMEM((CHUNK,), jnp.int32),
            pltpu.VMEM((N_TAB,), jnp.float32),
            pltpu.VMEM((CHUNK,), jnp.float32),
            pltpu.VMEM((CHUNK, F), jnp.float32),
            pltpu.VMEM((CHUNK, F), jnp.float32),
            pltpu.VMEM((CHUNK, F), jnp.float32),
            pltpu.VMEM((CHUNK, F), jnp.float32),
            pltpu.VMEM((CHUNK, F), jnp.float32),
            pltpu.VMEM((CHUNK, F), jnp.float32),
            pltpu.VMEM_SHARED((N_TAB, F), jnp.float32),
            pltpu.SemaphoreType.DMA,
            pltpu.SemaphoreType.DMA,
            pltpu.SemaphoreType.DMA,
            pltpu.SemaphoreType.DMA,
            pltpu.SemaphoreType.DMA,
            pltpu.SemaphoreType.DMA,
            pltpu.SemaphoreType.DMA,
            pltpu.SemaphoreType.DMA,
            pltpu.SemaphoreType.DMA,
        ],
        compiler_params=pltpu.CompilerParams(needs_layout_passes=False,
                                             use_tc_tiling_on_sc=False,
                                             disable_bounds_checks=True,
                                             disable_semaphore_checks=True),
    )(_sc_body)
    return fn(x, y, c, a_flat)


def kernel(x, y, batch_c, a):
    return _run(x, y.astype(jnp.int32), batch_c, a.reshape(-1))


# speedup vs baseline: 1.0716x; 1.0716x over previous
import functools

import jax
import jax.numpy as jnp
from jax import lax
from jax.experimental import pallas as pl
from jax.experimental.pallas import tpu as pltpu
from jax.experimental.pallas import tpu_sc as plsc

B = 16384
F = 128
N_TAB = 1000

NC = 2
NS = 16
NW = NC * NS
ROWS_PER_W = B // NW
CHUNK = 64
NCHUNK = ROWS_PER_W // CHUNK
NBUF = 3
LANES = 16
GROUPS = CHUNK // LANES

STAGE_TILES = 8
STAGE_ROWS = N_TAB // STAGE_TILES


def _sc_body(x_hbm, y_hbm, c_hbm, a_hbm, out_hbm,
             idxb, a_tab, avb, xb, cb, ob, c_spm,
             sem_a, sem_i, sem_s, sxa, sca, soa):
    sid = lax.axis_index("s")
    wid = sid * NC + lax.axis_index("c")
    base = wid * ROWS_PER_W

    def x_copy(j, slot):
        return pltpu.make_async_copy(
            x_hbm.at[pl.ds(base + j * CHUNK, CHUNK), :], xb.at[slot],
            sxa.at[slot])

    def c_copy(j, slot):
        return pltpu.make_async_copy(
            c_spm.at[idxb.at[j]], cb.at[slot], sca.at[slot])

    def out_copy(j, slot):
        return pltpu.make_async_copy(
            ob.at[slot], out_hbm.at[pl.ds(base + j * CHUNK, CHUNK), :],
            soa.at[slot])

    cp_a = pltpu.async_copy(a_hbm, a_tab, sem_a)

    @pl.when(sid < STAGE_TILES)
    def _stage():
        pltpu.async_copy(
            c_hbm.at[pl.ds(sid * STAGE_ROWS, STAGE_ROWS), :],
            c_spm.at[pl.ds(sid * STAGE_ROWS, STAGE_ROWS), :],
            sem_s).wait()

    cpi = [pltpu.async_copy(y_hbm.at[pl.ds(base + s * CHUNK, CHUNK)],
                            idxb.at[s], sem_i)
           for s in range(NCHUNK)]
    for cp in cpi:
        cp.wait()
    plsc.subcore_barrier()

    for j in range(NBUF):
        x_copy(j, j).start()
        c_copy(j, j).start()
    cp_a.wait()

    def chunk_step(j, carry):
        slot = lax.rem(j, NBUF)
        x_copy(j, slot).wait()
        c_copy(j, slot).wait()

        @pl.when(j >= NBUF)
        def _drain():
            out_copy(j - NBUF, slot).wait()

        @plsc.parallel_loop(0, GROUPS, step=1, unroll=1)
        def scale_body(g):
            idxg = idxb[j, pl.ds(g * LANES, LANES)]
            avb[pl.ds(g * LANES, LANES)] = plsc.load_gather(a_tab, [idxg])

        @plsc.parallel_loop(0, CHUNK, step=1, unroll=2)
        def row_body(row):
            sv = plsc.load_gather(
                avb, [jnp.full((LANES,), row, dtype=jnp.int32)])
            for k in range(F // LANES):
                sl = pl.ds(k * LANES, LANES)
                ob[slot, row, sl] = xb[slot, row, sl] * sv + cb[slot, row, sl]

        out_copy(j, slot).start()
        nj = j + NBUF

        @pl.when(nj < NCHUNK)
        def _prefetch():
            x_copy(nj, slot).start()
            c_copy(nj, slot).start()

        return carry

    lax.fori_loop(0, NCHUNK, chunk_step, 0)

    for j in range(NCHUNK - NBUF, NCHUNK):
        out_copy(j, j % NBUF).wait()


@jax.jit
def _run(x, y, c, a_flat):
    mesh = plsc.VectorSubcoreMesh(core_axis_name="c", subcore_axis_name="s")
    fn = functools.partial(
        pl.kernel,
        out_type=jax.ShapeDtypeStruct((B, F), jnp.float32),
        mesh=mesh,
        scratch_types=[
            pltpu.VMEM((NCHUNK, CHUNK), jnp.int32),
            pltpu.VMEM((N_TAB,), jnp.float32),
            pltpu.VMEM((CHUNK,), jnp.float32),
            pltpu.VMEM((NBUF, CHUNK, F), jnp.float32),
            pltpu.VMEM((NBUF, CHUNK, F), jnp.float32),
            pltpu.VMEM((NBUF, CHUNK, F), jnp.float32),
            pltpu.VMEM_SHARED((N_TAB, F), jnp.float32),
            pltpu.SemaphoreType.DMA,
            pltpu.SemaphoreType.DMA,
            pltpu.SemaphoreType.DMA,
            pltpu.SemaphoreType.DMA((NBUF,)),
            pltpu.SemaphoreType.DMA((NBUF,)),
            pltpu.SemaphoreType.DMA((NBUF,)),
        ],
        compiler_params=pltpu.CompilerParams(needs_layout_passes=False,
                                             use_tc_tiling_on_sc=False),
    )(_sc_body)
    return fn(x, y, c, a_flat)


def kernel(x, y, batch_c, a):
    return _run(x, y.astype(jnp.int32), batch_c, a.reshape(-1))


# speedup vs baseline: 1.0749x; 1.0030x over previous
import functools

import jax
import jax.numpy as jnp
from jax import lax
from jax.experimental import pallas as pl
from jax.experimental.pallas import tpu as pltpu
from jax.experimental.pallas import tpu_sc as plsc

B = 16384
F = 128
N_TAB = 1000

NC = 2
NS = 16
NW = NC * NS
ROWS_PER_W = B // NW
CHUNK = 64
NCHUNK = ROWS_PER_W // CHUNK
NBUF = 4
LANES = 16
GROUPS = CHUNK // LANES

STAGE_TILES = 8
STAGE_ROWS = N_TAB // STAGE_TILES


def _sc_body(x_hbm, y_hbm, c_hbm, a_hbm, out_hbm,
             idxb, a_tab, avb, xb, cb, ob, c_spm,
             sem_a, sem_i, sem_s, sxa, sca, soa):
    sid = lax.axis_index("s")
    wid = sid * NC + lax.axis_index("c")
    base = wid * ROWS_PER_W

    def x_copy(j, slot):
        return pltpu.make_async_copy(
            x_hbm.at[pl.ds(base + j * CHUNK, CHUNK), :], xb.at[slot],
            sxa.at[slot])

    def c_copy(j, slot):
        return pltpu.make_async_copy(
            c_spm.at[idxb.at[j]], cb.at[slot], sca.at[slot])

    def out_copy(j, slot):
        return pltpu.make_async_copy(
            ob.at[slot], out_hbm.at[pl.ds(base + j * CHUNK, CHUNK), :],
            soa.at[slot])

    cp_a = pltpu.async_copy(a_hbm, a_tab, sem_a)

    @pl.when(sid < STAGE_TILES)
    def _stage():
        pltpu.async_copy(
            c_hbm.at[pl.ds(sid * STAGE_ROWS, STAGE_ROWS), :],
            c_spm.at[pl.ds(sid * STAGE_ROWS, STAGE_ROWS), :],
            sem_s).wait()

    cpi = [pltpu.async_copy(y_hbm.at[pl.ds(base + s * CHUNK, CHUNK)],
                            idxb.at[s], sem_i)
           for s in range(NCHUNK)]
    for cp in cpi:
        cp.wait()
    plsc.subcore_barrier()

    for j in range(NBUF):
        x_copy(j, j).start()
        c_copy(j, j).start()
    cp_a.wait()

    def chunk_step(j, carry):
        slot = lax.rem(j, NBUF)
        x_copy(j, slot).wait()
        c_copy(j, slot).wait()

        @pl.when(j >= NBUF)
        def _drain():
            out_copy(j - NBUF, slot).wait()

        @plsc.parallel_loop(0, GROUPS, step=1, unroll=1)
        def scale_body(g):
            idxg = idxb[j, pl.ds(g * LANES, LANES)]
            avb[pl.ds(g * LANES, LANES)] = plsc.load_gather(a_tab, [idxg])

        @plsc.parallel_loop(0, CHUNK, step=1, unroll=4)
        def row_body(row):
            sv = plsc.load_gather(
                avb, [jnp.full((LANES,), row, dtype=jnp.int32)])
            for k in range(F // LANES):
                sl = pl.ds(k * LANES, LANES)
                ob[slot, row, sl] = xb[slot, row, sl] * sv + cb[slot, row, sl]

        out_copy(j, slot).start()
        nj = j + NBUF

        @pl.when(nj < NCHUNK)
        def _prefetch():
            x_copy(nj, slot).start()
            c_copy(nj, slot).start()

        return carry

    lax.fori_loop(0, NCHUNK, chunk_step, 0)

    for j in range(NCHUNK - NBUF, NCHUNK):
        out_copy(j, j % NBUF).wait()


@jax.jit
def _run(x, y, c, a_flat):
    mesh = plsc.VectorSubcoreMesh(core_axis_name="c", subcore_axis_name="s")
    fn = functools.partial(
        pl.kernel,
        out_type=jax.ShapeDtypeStruct((B, F), jnp.float32),
        mesh=mesh,
        scratch_types=[
            pltpu.VMEM((NCHUNK, CHUNK), jnp.int32),
            pltpu.VMEM((N_TAB,), jnp.float32),
            pltpu.VMEM((CHUNK,), jnp.float32),
            pltpu.VMEM((NBUF, CHUNK, F), jnp.float32),
            pltpu.VMEM((NBUF, CHUNK, F), jnp.float32),
            pltpu.VMEM((NBUF, CHUNK, F), jnp.float32),
            pltpu.VMEM_SHARED((N_TAB, F), jnp.float32),
            pltpu.SemaphoreType.DMA,
            pltpu.SemaphoreType.DMA,
            pltpu.SemaphoreType.DMA,
            pltpu.SemaphoreType.DMA((NBUF,)),
            pltpu.SemaphoreType.DMA((NBUF,)),
            pltpu.SemaphoreType.DMA((NBUF,)),
        ],
        compiler_params=pltpu.CompilerParams(needs_layout_passes=False,
                                             use_tc_tiling_on_sc=False),
    )(_sc_body)
    return fn(x, y, c, a_flat)


def kernel(x, y, batch_c, a):
    return _run(x, y.astype(jnp.int32), batch_c, a.reshape(-1))
